# baseline (speedup 1.0000x reference)
import functools

import jax
import jax.numpy as jnp
from jax import lax
from jax.experimental import pallas as pl
from jax.experimental.pallas import tpu as pltpu
from jax.experimental.pallas import tpu_sc as plsc

_INFO = plsc.get_sparse_core_info()
_NC = _INFO.num_cores
_NS = _INFO.num_subcores
_NW = _NC * _NS

_CB = 256


@functools.partial(jax.jit, static_argnums=(2, 3, 4))
def _sc_gather_t(table, xT3, n_fields, batch, embed_dim):
    n_bt = batch // 128
    tasks_per_f = batch // _CB
    n_tasks = n_fields * tasks_per_f
    per_w = n_tasks // _NW
    tr_n = _CB * embed_dim
    plane = embed_dim * batch
    rt_stride = 8 * batch
    n_flat = n_fields * plane

    mesh = plsc.VectorSubcoreMesh(core_axis_name="c", subcore_axis_name="s")

    @functools.partial(
        pl.kernel,
        mesh=mesh,
        out_type=jax.ShapeDtypeStruct((n_flat,), jnp.float32),
        scratch_types=[
            pltpu.VMEM((2, 2, 128), jnp.int32),
            pltpu.VMEM((2, _CB, embed_dim), jnp.float32),
            pltpu.VMEM((2, tr_n), jnp.float32),
            pltpu.SemaphoreType.DMA,
            pltpu.SemaphoreType.DMA,
            pltpu.SemaphoreType.DMA,
            pltpu.SemaphoreType.DMA,
        ],
        compiler_params=pltpu.CompilerParams(
            use_tc_tiling_on_sc=False, needs_layout_passes=False
        ),
    )
    def k(table_hbm, idx_hbm, out_hbm, idx_v, rows_v, tr_v, g0, g1, o0, o1):
        wid = lax.axis_index("s") * _NC + lax.axis_index("c")
        gsems = (g0, g1)
        osems = (o0, o1)
        lane = lax.iota(jnp.int32, 16)
        perm = [lax.rem(lane + d, 16) for d in range(16)]
        d_in = [lane * embed_dim + perm[d] for d in range(16)]
        d_out = [
            (perm[d] // 8) * 2048 + lax.rem(perm[d], 8) * 128 + lane
            for d in range(16)
        ]

        def task_of(kk):
            tid = wid * per_w + kk
            return tid // tasks_per_f, lax.rem(tid, tasks_per_f)

        def load_and_fire(kk, slot):
            f, t = task_of(kk)
            pltpu.sync_copy(idx_hbm.at[f, pl.ds(2 * t, 2)], idx_v.at[slot])
            for jj in range(2):
                pltpu.async_copy(
                    table_hbm.at[idx_v.at[slot, jj]],
                    rows_v.at[slot, pl.ds(jj * 128, 128)],
                    gsems[slot],
                )

        def drain(sem, slot):
            pltpu.make_async_copy(
                table_hbm.at[pl.ds(0, _CB)], rows_v.at[slot], sem
            ).wait()

        def do_task(kk, slot):
            @pl.when(kk + 1 < per_w)
            def _():
                load_and_fire(kk + 1, 1 - slot)

            drain(gsems[slot], slot)

            @pl.when(kk >= 2)
            def _():
                drain(osems[slot], slot)

            rows2d = rows_v.at[slot]
            tr_flat = tr_v.at[slot]

            def trans_blk(b16, c2):
                rvec = lane + b16 * 16
                ob = (b16 // 8) * 1024 + lax.rem(b16, 8) * 16
                for c0 in range(0, embed_dim, 16):
                    s_out = ob + (c0 // 8) * 2048
                    for d in range(16):
                        v = plsc.load_gather(rows2d, [rvec, perm[d] + c0])
                        plsc.store_scatter(tr_flat, [d_out[d] + s_out], v)
                return c2

            lax.fori_loop(0, _CB // 16, trans_blk, 0, unroll=4)

            f, t = task_of(kk)
            obase = f * plane + t * 2048
            for rt in range(8):
                pltpu.async_copy(
                    tr_v.at[slot, pl.ds(rt * 2048, 2048)],
                    out_hbm.at[pl.ds(obase + rt * rt_stride, 2048)],
                    osems[slot],
                )

        load_and_fire(0, 0)

        def body(kk2, carry):
            for slot in range(2):
                do_task(2 * kk2 + slot, slot)
            return carry

        lax.fori_loop(0, per_w // 2, body, 0)
        drain(osems[0], 0)
        drain(osems[1], 1)

    return k(table, xT3)


def kernel(x, table):
    batch, n_fields = x.shape
    embed_dim = table.shape[1]
    xT3 = x.astype(jnp.int32).T.reshape(n_fields, batch // 128, 128)
    flat = _sc_gather_t(table, xT3, n_fields, batch, embed_dim)
    out5 = flat.reshape(n_fields, embed_dim // 8, batch // 128, 8, 128)
    plane = out5.transpose(0, 1, 3, 2, 4).reshape(n_fields, embed_dim, batch)
    return plane.transpose(2, 0, 1)


# speedup vs baseline: 1.0133x; 1.0133x over previous
import functools

import jax
import jax.numpy as jnp
from jax import lax
from jax.experimental import pallas as pl
from jax.experimental.pallas import tpu as pltpu
from jax.experimental.pallas import tpu_sc as plsc

_INFO = plsc.get_sparse_core_info()
_NC = _INFO.num_cores
_NS = _INFO.num_subcores
_NW = _NC * _NS

_CB = 256


@functools.partial(jax.jit, static_argnums=(2, 3, 4))
def _sc_gather_t(table, xT3, n_fields, batch, embed_dim):
    n_bt = batch // 128
    tasks_per_f = batch // _CB
    n_tasks = n_fields * tasks_per_f
    per_w = n_tasks // _NW
    tr_n = _CB * embed_dim
    plane = embed_dim * batch
    rt_stride = 8 * batch
    n_flat = n_fields * plane

    mesh = plsc.VectorSubcoreMesh(core_axis_name="c", subcore_axis_name="s")

    @functools.partial(
        pl.kernel,
        mesh=mesh,
        out_type=jax.ShapeDtypeStruct((n_flat,), jnp.float32),
        scratch_types=[
            pltpu.VMEM((2, 2, 128), jnp.int32),
            pltpu.VMEM((2, _CB, embed_dim), jnp.float32),
            pltpu.VMEM((2, tr_n), jnp.float32),
            pltpu.SemaphoreType.DMA,
            pltpu.SemaphoreType.DMA,
            pltpu.SemaphoreType.DMA,
            pltpu.SemaphoreType.DMA,
        ],
        compiler_params=pltpu.CompilerParams(
            use_tc_tiling_on_sc=False, needs_layout_passes=False
        ),
    )
    def k(table_hbm, idx_hbm, out_hbm, idx_v, rows_v, tr_v, g0, g1, o0, o1):
        wid = lax.axis_index("s") * _NC + lax.axis_index("c")
        gsems = (g0, g1)
        osems = (o0, o1)
        lane = lax.iota(jnp.int32, 16)
        perm = [lax.rem(lane + d, 16) for d in range(16)]
        d_out = [
            (perm[d] // 8) * 2048 + lax.rem(perm[d], 8) * 128 + lane
            for d in range(16)
        ]

        def task_of(kk):
            tid = wid * per_w + kk
            return tid // tasks_per_f, lax.rem(tid, tasks_per_f)

        def load_and_fire(kk, slot):
            f, t = task_of(kk)
            pltpu.sync_copy(idx_hbm.at[f, pl.ds(2 * t, 2)], idx_v.at[slot])
            for jj in range(2):
                pltpu.async_copy(
                    table_hbm.at[idx_v.at[slot, jj]],
                    rows_v.at[slot, pl.ds(jj * 128, 128)],
                    gsems[slot],
                )

        def drain(sem, slot):
            pltpu.make_async_copy(
                table_hbm.at[pl.ds(0, _CB)], rows_v.at[slot], sem
            ).wait()

        def do_task(kk, slot):
            @pl.when(kk + 1 < per_w)
            def _():
                load_and_fire(kk + 1, 1 - slot)

            drain(gsems[slot], slot)

            @pl.when(kk >= 2)
            def _():
                drain(osems[slot], slot)

            rows2d = rows_v.at[slot]
            tr_flat = tr_v.at[slot]

            def trans_blk(b16, c2):
                rvec = lane + b16 * 16
                ob = (b16 // 8) * 1024 + lax.rem(b16, 8) * 16
                for c0 in range(0, embed_dim, 16):
                    s_out = ob + (c0 // 8) * 2048
                    for d in range(16):
                        v = plsc.load_gather(rows2d, [rvec, perm[d] + c0])
                        plsc.store_scatter(tr_flat, [d_out[d] + s_out], v)
                return c2

            lax.fori_loop(0, _CB // 16, trans_blk, 0, unroll=2)

            f, t = task_of(kk)
            obase = f * plane + t * 2048
            for rt in range(8):
                pltpu.async_copy(
                    tr_v.at[slot, pl.ds(rt * 2048, 2048)],
                    out_hbm.at[pl.ds(obase + rt * rt_stride, 2048)],
                    osems[slot],
                )

        load_and_fire(0, 0)

        def body(kk2, carry):
            for slot in range(2):
                do_task(2 * kk2 + slot, slot)
            return carry

        lax.fori_loop(0, per_w // 2, body, 0)
        drain(osems[0], 0)
        drain(osems[1], 1)

    return k(table, xT3)


def kernel(x, table):
    batch, n_fields = x.shape
    embed_dim = table.shape[1]
    xT3 = x.astype(jnp.int32).T.reshape(n_fields, batch // 128, 128)
    flat = _sc_gather_t(table, xT3, n_fields, batch, embed_dim)
    out5 = flat.reshape(n_fields, embed_dim // 8, batch // 128, 8, 128)
    plane = out5.transpose(0, 1, 3, 2, 4).reshape(n_fields, embed_dim, batch)
    return plane.transpose(2, 0, 1)
